# baseline (speedup 1.0000x reference)
import jax
import jax.numpy as jnp
from jax.experimental import pallas as pl
from jax.experimental.pallas import tpu as pltpu

BN = 512
NCHUNK = 4


def _mm_kernel(scale_ref, x_hbm, w_ref, o_ref, x_vmem, xb_vmem, sems):
    n = pl.program_id(0)
    ch = x_vmem.shape[0] // NCHUNK

    wb = w_ref[...].astype(jnp.bfloat16)

    def _dot(xs):
        return jax.lax.dot_general(
            xs,
            wb,
            (((1,), (1,)), ((), ())),
            preferred_element_type=jnp.float32,
        ) * scale_ref[0]

    def _copy(c):
        return pltpu.make_async_copy(
            x_hbm.at[pl.ds(c * ch, ch), :],
            x_vmem.at[pl.ds(c * ch, ch), :],
            sems.at[c],
        )

    @pl.when(n == 0)
    def _():
        for c in range(NCHUNK):
            _copy(c).start()
        for c in range(NCHUNK):
            _copy(c).wait()
            rows = pl.ds(c * ch, ch)
            xb_vmem[rows, :] = x_vmem[rows, :].astype(jnp.bfloat16)
            o_ref[rows, :] = _dot(xb_vmem[rows, :])

    @pl.when(n > 0)
    def _():
        o_ref[...] = _dot(xb_vmem[...])


@jax.jit
def kernel(x, W, scale):
    M, K = x.shape
    N = W.shape[0]
    return pl.pallas_call(
        _mm_kernel,
        grid_spec=pltpu.PrefetchScalarGridSpec(
            num_scalar_prefetch=1,
            grid=(N // BN,),
            in_specs=[
                pl.BlockSpec(memory_space=pl.ANY),
                pl.BlockSpec((BN, K), lambda n, *_: (n, 0)),
            ],
            out_specs=pl.BlockSpec((M, BN), lambda n, *_: (0, n)),
            scratch_shapes=[
                pltpu.VMEM((M, K), jnp.float32),
                pltpu.VMEM((M, K), jnp.bfloat16),
                pltpu.SemaphoreType.DMA((NCHUNK,)),
            ],
        ),
        out_shape=jax.ShapeDtypeStruct((M, N), jnp.float32),
        compiler_params=pltpu.CompilerParams(
            dimension_semantics=("arbitrary",),
            vmem_limit_bytes=100 * 1024 * 1024,
        ),
    )(scale, x, W)
